# baseline (speedup 1.0000x reference)
import functools

import jax
import jax.numpy as jnp
from jax import lax
from jax.experimental import pallas as pl
from jax.experimental.pallas import tpu as pltpu
from jax.experimental.pallas import tpu_sc as plsc

N, D, E, K, H, O = 2048, 768, 8, 2, 768, 768
TB = 128
NB = N // TB
LANES = 128
TILE = 256
TSHIFT = TILE.bit_length() - 1
T = (N * K + E * (TILE - 1)) // TILE
S = T * TILE
NW = 32
CHUNK = N // NW
NEG = -1e30


def _router_body(x_ref, wg_ref, g0_ref, g1_ref, pos0_ref, pos1_ref,
                 eot_ref, loss_ref, cnt_ref, imp_ref, load_ref,
                 a1_s, a2_s, r0_s, r1_s):
    i = pl.program_id(0)

    @pl.when(i == 0)
    def _init():
        cnt_ref[...] = jnp.zeros((1, LANES), jnp.float32)
        imp_ref[...] = jnp.zeros((1, LANES), jnp.float32)
        load_ref[...] = jnp.zeros((1, LANES), jnp.float32)

    xb = x_ref[...]
    logits = jnp.dot(xb, wg_ref[...], preferred_element_type=jnp.float32)
    lane = lax.broadcasted_iota(jnp.int32, (TB, LANES), 1)
    logits = jnp.where(lane < E, logits, NEG)

    l1 = jnp.max(logits, axis=1, keepdims=True)
    m1 = jnp.where(logits >= l1, LANES - 1 - lane, -1)
    a1 = (LANES - 1) - jnp.max(m1, axis=1, keepdims=True)
    oh1 = lane == a1
    logits2 = jnp.where(oh1, NEG, logits)
    l2 = jnp.max(logits2, axis=1, keepdims=True)
    m2 = jnp.where(logits2 >= l2, LANES - 1 - lane, -1)
    a2 = (LANES - 1) - jnp.max(m2, axis=1, keepdims=True)
    oh2 = lane == a2

    dexp = jnp.exp(l2 - l1)
    g1 = 1.0 / (1.0 + dexp)
    g2 = dexp / (1.0 + dexp)

    oh1f = oh1.astype(jnp.float32)
    oh2f = oh2.astype(jnp.float32)
    pair = oh1f + oh2f
    row = lax.broadcasted_iota(jnp.int32, (TB, TB), 0)
    col = lax.broadcasted_iota(jnp.int32, (TB, TB), 1)
    tri = (col < row).astype(jnp.bfloat16)
    cum_prev = jnp.dot(tri, pair.astype(jnp.bfloat16),
                       preferred_element_type=jnp.float32)
    base = cnt_ref[...]
    r0 = jnp.sum(oh1f * (base + cum_prev), axis=1, keepdims=True)
    r1 = jnp.sum(oh2f * (base + cum_prev + oh1f), axis=1, keepdims=True)
    cnt_ref[...] = base + jnp.sum(pair, axis=0, keepdims=True)
    imp_ref[...] = imp_ref[...] + jnp.sum(g1 * oh1f + g2 * oh2f, axis=0,
                                          keepdims=True)
    load_ref[...] = load_ref[...] + jnp.sum(
        (g1 > 0).astype(jnp.float32) * oh1f
        + (g2 > 0).astype(jnp.float32) * oh2f, axis=0, keepdims=True)

    eye = (row == col).astype(jnp.float32)

    def col2row(v_col):
        return jnp.sum(eye * v_col, axis=0, keepdims=True)

    a1_s[pl.ds(i, 1), :] = col2row(a1.astype(jnp.float32))
    a2_s[pl.ds(i, 1), :] = col2row(a2.astype(jnp.float32))
    r0_s[pl.ds(i, 1), :] = col2row(r0)
    r1_s[pl.ds(i, 1), :] = col2row(r1)
    g0_ref[...] = col2row(g1).reshape(1, 1, LANES)
    g1_ref[...] = col2row(g2).reshape(1, 1, LANES)

    @pl.when(i == NB - 1)
    def _finalize():
        cnt_i = cnt_ref[...].astype(jnp.int32)
        pc = ((cnt_i + (TILE - 1)) >> TSHIFT) << TSHIFT
        pc_f = pc.astype(jnp.bfloat16)
        mstrict = (row < col).astype(jnp.bfloat16)
        off_f = jnp.dot(pc_f, mstrict, preferred_element_type=jnp.float32)
        off_i = off_f.astype(jnp.int32)

        for b in range(NB):
            a1row = a1_s[pl.ds(b, 1), :].astype(jnp.int32)
            a2row = a2_s[pl.ds(b, 1), :].astype(jnp.int32)
            m1 = (jnp.broadcast_to(a1row, (LANES, LANES)) == row)
            m2 = (jnp.broadcast_to(a2row, (LANES, LANES)) == row)
            off_b = off_f.astype(jnp.bfloat16)
            p0 = jnp.dot(off_b, m1.astype(jnp.bfloat16),
                         preferred_element_type=jnp.float32) + r0_s[
                             pl.ds(b, 1), :]
            p1 = jnp.dot(off_b, m2.astype(jnp.bfloat16),
                         preferred_element_type=jnp.float32) + r1_s[
                             pl.ds(b, 1), :]
            pos0_ref[b] = p0.astype(jnp.int32)
            pos1_ref[b] = p1.astype(jnp.int32)

        tstart = lax.broadcasted_iota(jnp.int32, (TB, LANES), 0) * TILE
        offb = jnp.broadcast_to(off_i, (TB, LANES))
        cond = ((offb <= tstart) & (lane < E)).astype(jnp.float32)
        eot_col = jnp.clip(jnp.sum(cond, axis=1, keepdims=True) - 1.0,
                           0.0, float(E - 1))
        eot_ref[...] = col2row(eot_col).astype(jnp.int32).reshape(
            1, 1, LANES)

        lane_row = lane[:1, :]
        imp = imp_ref[...]
        lod = load_ref[...]

        def cv(v):
            mean = jnp.sum(v, axis=1, keepdims=True) / E
            var = jnp.sum(jnp.where(lane_row < E, v - mean, 0.0) ** 2,
                          axis=1, keepdims=True) / (E - 1)
            return var / (mean * mean + 1e-10)

        loss = (cv(imp) + cv(lod)) * 0.01
        loss_ref[...] = jnp.broadcast_to(loss, (1, LANES)).reshape(
            1, 1, LANES)


def _dispatch(pos0_hbm, pos1_hbm, x_hbm, xs_hbm,
              pos0_v, pos1_v, x_v, sem0, sem1):
    wid = lax.axis_index("s") * 2 + lax.axis_index("c")
    base = wid * CHUNK
    pltpu.sync_copy(pos0_hbm.at[pl.ds(base, CHUNK)], pos0_v)
    pltpu.sync_copy(pos1_hbm.at[pl.ds(base, CHUNK)], pos1_v)
    pltpu.sync_copy(x_hbm.at[pl.ds(base, CHUNK)], x_v)
    c0 = pltpu.async_copy(x_v, xs_hbm.at[pos0_v], sem0)
    c1 = pltpu.async_copy(x_v, xs_hbm.at[pos1_v], sem1)
    c0.wait()
    c1.wait()


def _combine(os_hbm, pos0_hbm, pos1_hbm, g0_hbm, g1_hbm,
             y_hbm, pos0_v, pos1_v, g0_v, g1_v, a_v, b_v, sem0, sem1):
    wid = lax.axis_index("s") * 2 + lax.axis_index("c")
    base = wid * CHUNK
    pltpu.sync_copy(pos0_hbm.at[pl.ds(base, CHUNK)], pos0_v)
    pltpu.sync_copy(pos1_hbm.at[pl.ds(base, CHUNK)], pos1_v)
    pltpu.sync_copy(g0_hbm.at[pl.ds(base, CHUNK)], g0_v.at[pl.ds(0, CHUNK)])
    pltpu.sync_copy(g1_hbm.at[pl.ds(base, CHUNK)], g1_v.at[pl.ds(0, CHUNK)])
    c0 = pltpu.async_copy(os_hbm.at[pos0_v], a_v, sem0)
    c1 = pltpu.async_copy(os_hbm.at[pos1_v], b_v, sem1)
    c0.wait()
    c1.wait()

    def body(r, carry):
        ga = g0_v[pl.ds(r, 16)][0]
        gb = g1_v[pl.ds(r, 16)][0]
        for j in range(O // 16):
            sl = (r, pl.ds(j * 16, 16))
            a_v[sl] = ga * a_v[sl] + gb * b_v[sl]
        return carry

    lax.fori_loop(0, CHUNK, body, 0)
    pltpu.sync_copy(a_v, y_hbm.at[pl.ds(base, CHUNK)])


@functools.lru_cache(maxsize=None)
def _sc_kernels():
    mesh = plsc.VectorSubcoreMesh(core_axis_name="c", subcore_axis_name="s")
    dispatch = functools.partial(
        pl.kernel,
        out_type=jax.ShapeDtypeStruct((S, D), jnp.float32),
        mesh=mesh,
        scratch_types=[pltpu.VMEM((CHUNK,), jnp.int32),
                       pltpu.VMEM((CHUNK,), jnp.int32),
                       pltpu.VMEM((CHUNK, D), jnp.float32),
                       pltpu.SemaphoreType.DMA,
                       pltpu.SemaphoreType.DMA],
    )(_dispatch)
    combine = functools.partial(
        pl.kernel,
        out_type=jax.ShapeDtypeStruct((N, O), jnp.float32),
        mesh=mesh,
        scratch_types=[pltpu.VMEM((CHUNK,), jnp.int32),
                       pltpu.VMEM((CHUNK,), jnp.int32),
                       pltpu.VMEM((CHUNK + 16,), jnp.float32),
                       pltpu.VMEM((CHUNK + 16,), jnp.float32),
                       pltpu.VMEM((CHUNK, O), jnp.float32),
                       pltpu.VMEM((CHUNK, O), jnp.float32),
                       pltpu.SemaphoreType.DMA,
                       pltpu.SemaphoreType.DMA],
    )(_combine)
    return dispatch, combine


def _ffn_body(eot_ref, xs_ref, w1_ref, b1_ref, w2_ref, b2_ref, out_ref):
    xb = xs_ref[...].astype(jnp.bfloat16)
    w1 = w1_ref[0].astype(jnp.bfloat16)
    w2 = w2_ref[0].astype(jnp.bfloat16)
    h = jnp.dot(xb, w1, preferred_element_type=jnp.float32)
    h = jnp.maximum(h + b1_ref[0], 0.0).astype(jnp.bfloat16)
    out_ref[...] = jnp.dot(h, w2,
                           preferred_element_type=jnp.float32) + b2_ref[0]


_ffn_grid = pltpu.PrefetchScalarGridSpec(
    num_scalar_prefetch=1,
    grid=(T,),
    in_specs=[
        pl.BlockSpec((TILE, D), lambda t, eot: (t, 0)),
        pl.BlockSpec((1, D, H), lambda t, eot: (eot[t], 0, 0)),
        pl.BlockSpec((1, 1, H), lambda t, eot: (eot[t], 0, 0)),
        pl.BlockSpec((1, H, O), lambda t, eot: (eot[t], 0, 0)),
        pl.BlockSpec((1, 1, O), lambda t, eot: (eot[t], 0, 0)),
    ],
    out_specs=pl.BlockSpec((TILE, O), lambda t, eot: (t, 0)),
)


def kernel(x, w_gate, w_noise, W1, b1, W2, b2):
    del w_noise
    wg_pad = jnp.zeros((D, LANES), jnp.float32).at[:, :E].set(w_gate)

    (g0o, g1o, pos0o, pos1o, eoto, losso) = pl.pallas_call(
        _router_body,
        grid=(NB,),
        in_specs=[pl.BlockSpec((TB, D), lambda i: (i, 0)),
                  pl.BlockSpec((D, LANES), lambda i: (0, 0))],
        out_specs=[pl.BlockSpec((1, 1, LANES), lambda i: (i, 0, 0))] * 2
        + [pl.BlockSpec((NB, 1, LANES), lambda i: (0, 0, 0))] * 2
        + [pl.BlockSpec((1, 1, LANES), lambda i: (0, 0, 0))] * 2,
        out_shape=[jax.ShapeDtypeStruct((NB, 1, LANES), jnp.float32)] * 2
        + [jax.ShapeDtypeStruct((NB, 1, LANES), jnp.int32)] * 2
        + [jax.ShapeDtypeStruct((1, 1, LANES), jnp.int32)]
        + [jax.ShapeDtypeStruct((1, 1, LANES), jnp.float32)],
        scratch_shapes=[pltpu.VMEM((1, LANES), jnp.float32)] * 3
        + [pltpu.VMEM((NB, LANES), jnp.float32)] * 4,
    )(x, wg_pad)

    g0 = g0o.reshape(N)
    g1 = g1o.reshape(N)
    eot = eoto.reshape(LANES)[:T]
    loss = losso.reshape(LANES)[0]
    pos0 = pos0o.reshape(N)
    pos1 = pos1o.reshape(N)

    dispatch, combine = _sc_kernels()
    x_sorted = dispatch(pos0, pos1, x)

    out_sorted = pl.pallas_call(
        _ffn_body,
        grid_spec=_ffn_grid,
        out_shape=jax.ShapeDtypeStruct((S, O), jnp.float32),
    )(eot, x_sorted, W1, b1.reshape(E, 1, H), W2, b2.reshape(E, 1, O))

    y = combine(out_sorted, pos0, pos1, g0, g1)
    return y, loss


# speedup vs baseline: 1.0026x; 1.0026x over previous
import functools

import jax
import jax.numpy as jnp
from jax import lax
from jax.experimental import pallas as pl
from jax.experimental.pallas import tpu as pltpu
from jax.experimental.pallas import tpu_sc as plsc

N, D, E, K, H, O = 2048, 768, 8, 2, 768, 768
TB = 128
NB = N // TB
LANES = 128
TILE = 256
TSHIFT = TILE.bit_length() - 1
T = (N * K + E * (TILE - 1)) // TILE
S = T * TILE
NW = 32
CHUNK = N // NW
NEG = -1e30


def _router_body(x_ref, wg_ref, g0_ref, g1_ref, pos0_ref, pos1_ref,
                 eot_ref, loss_ref, cnt_ref, imp_ref, load_ref,
                 a1_s, a2_s, r0_s, r1_s):
    i = pl.program_id(0)

    @pl.when(i == 0)
    def _init():
        cnt_ref[...] = jnp.zeros((1, LANES), jnp.float32)
        imp_ref[...] = jnp.zeros((1, LANES), jnp.float32)
        load_ref[...] = jnp.zeros((1, LANES), jnp.float32)

    xb = x_ref[...]
    logits = jnp.dot(xb, wg_ref[...], preferred_element_type=jnp.float32)
    lane = lax.broadcasted_iota(jnp.int32, (TB, LANES), 1)
    logits = jnp.where(lane < E, logits, NEG)

    l1 = jnp.max(logits, axis=1, keepdims=True)
    m1 = jnp.where(logits >= l1, LANES - 1 - lane, -1)
    a1 = (LANES - 1) - jnp.max(m1, axis=1, keepdims=True)
    oh1 = lane == a1
    logits2 = jnp.where(oh1, NEG, logits)
    l2 = jnp.max(logits2, axis=1, keepdims=True)
    m2 = jnp.where(logits2 >= l2, LANES - 1 - lane, -1)
    a2 = (LANES - 1) - jnp.max(m2, axis=1, keepdims=True)
    oh2 = lane == a2

    dexp = jnp.exp(l2 - l1)
    g1 = 1.0 / (1.0 + dexp)
    g2 = dexp / (1.0 + dexp)

    oh1f = oh1.astype(jnp.float32)
    oh2f = oh2.astype(jnp.float32)
    pair = oh1f + oh2f
    row = lax.broadcasted_iota(jnp.int32, (TB, TB), 0)
    col = lax.broadcasted_iota(jnp.int32, (TB, TB), 1)
    tri = (col < row).astype(jnp.bfloat16)
    cum_prev = jnp.dot(tri, pair.astype(jnp.bfloat16),
                       preferred_element_type=jnp.float32)
    base = cnt_ref[...]
    r0 = jnp.sum(oh1f * (base + cum_prev), axis=1, keepdims=True)
    r1 = jnp.sum(oh2f * (base + cum_prev + oh1f), axis=1, keepdims=True)
    cnt_ref[...] = base + jnp.sum(pair, axis=0, keepdims=True)
    imp_ref[...] = imp_ref[...] + jnp.sum(g1 * oh1f + g2 * oh2f, axis=0,
                                          keepdims=True)
    load_ref[...] = load_ref[...] + jnp.sum(
        (g1 > 0).astype(jnp.float32) * oh1f
        + (g2 > 0).astype(jnp.float32) * oh2f, axis=0, keepdims=True)

    eye = (row == col).astype(jnp.float32)

    def col2row(v_col):
        return jnp.sum(eye * v_col, axis=0, keepdims=True)

    a1_s[pl.ds(i, 1), :] = col2row(a1.astype(jnp.float32))
    a2_s[pl.ds(i, 1), :] = col2row(a2.astype(jnp.float32))
    r0_s[pl.ds(i, 1), :] = col2row(r0)
    r1_s[pl.ds(i, 1), :] = col2row(r1)
    g0_ref[...] = col2row(g1).reshape(1, 1, LANES)
    g1_ref[...] = col2row(g2).reshape(1, 1, LANES)

    @pl.when(i == NB - 1)
    def _finalize():
        cnt_i = cnt_ref[...].astype(jnp.int32)
        pc = ((cnt_i + (TILE - 1)) >> TSHIFT) << TSHIFT
        pc_f = pc.astype(jnp.bfloat16)
        mstrict = (row < col).astype(jnp.bfloat16)
        off_f = jnp.dot(pc_f, mstrict, preferred_element_type=jnp.float32)
        off_i = off_f.astype(jnp.int32)

        a1_all = a1_s[...]
        a2_all = a2_s[...]
        p0 = r0_s[...]
        p1 = r1_s[...]
        for e in range(E):
            oe = off_f[0, e]
            p0 = p0 + jnp.where(a1_all == e, oe, 0.0)
            p1 = p1 + jnp.where(a2_all == e, oe, 0.0)
        pos0_ref[...] = p0.astype(jnp.int32).reshape(NB, 1, LANES)
        pos1_ref[...] = p1.astype(jnp.int32).reshape(NB, 1, LANES)

        tstart = lax.broadcasted_iota(jnp.int32, (TB, LANES), 0) * TILE
        offb = jnp.broadcast_to(off_i, (TB, LANES))
        cond = ((offb <= tstart) & (lane < E)).astype(jnp.float32)
        eot_col = jnp.clip(jnp.sum(cond, axis=1, keepdims=True) - 1.0,
                           0.0, float(E - 1))
        eot_ref[...] = col2row(eot_col).astype(jnp.int32).reshape(
            1, 1, LANES)

        lane_row = lane[:1, :]
        imp = imp_ref[...]
        lod = load_ref[...]

        def cv(v):
            mean = jnp.sum(v, axis=1, keepdims=True) / E
            var = jnp.sum(jnp.where(lane_row < E, v - mean, 0.0) ** 2,
                          axis=1, keepdims=True) / (E - 1)
            return var / (mean * mean + 1e-10)

        loss = (cv(imp) + cv(lod)) * 0.01
        loss_ref[...] = jnp.broadcast_to(loss, (1, LANES)).reshape(
            1, 1, LANES)


def _dispatch(pos0_hbm, pos1_hbm, x_hbm, xs_hbm,
              pos0_v, pos1_v, x_v, sem0, sem1):
    wid = lax.axis_index("s") * 2 + lax.axis_index("c")
    base = wid * CHUNK
    pltpu.sync_copy(pos0_hbm.at[pl.ds(base, CHUNK)], pos0_v)
    pltpu.sync_copy(pos1_hbm.at[pl.ds(base, CHUNK)], pos1_v)
    pltpu.sync_copy(x_hbm.at[pl.ds(base, CHUNK)], x_v)
    c0 = pltpu.async_copy(x_v, xs_hbm.at[pos0_v], sem0)
    c1 = pltpu.async_copy(x_v, xs_hbm.at[pos1_v], sem1)
    c0.wait()
    c1.wait()


def _combine(os_hbm, pos0_hbm, pos1_hbm, g0_hbm, g1_hbm,
             y_hbm, pos0_v, pos1_v, g0_v, g1_v, a_v, b_v, sem0, sem1):
    wid = lax.axis_index("s") * 2 + lax.axis_index("c")
    base = wid * CHUNK
    pltpu.sync_copy(pos0_hbm.at[pl.ds(base, CHUNK)], pos0_v)
    pltpu.sync_copy(pos1_hbm.at[pl.ds(base, CHUNK)], pos1_v)
    pltpu.sync_copy(g0_hbm.at[pl.ds(base, CHUNK)], g0_v.at[pl.ds(0, CHUNK)])
    pltpu.sync_copy(g1_hbm.at[pl.ds(base, CHUNK)], g1_v.at[pl.ds(0, CHUNK)])
    c0 = pltpu.async_copy(os_hbm.at[pos0_v], a_v, sem0)
    c1 = pltpu.async_copy(os_hbm.at[pos1_v], b_v, sem1)
    c0.wait()
    c1.wait()

    def body(r, carry):
        ga = g0_v[pl.ds(r, 16)][0]
        gb = g1_v[pl.ds(r, 16)][0]
        for j in range(O // 16):
            sl = (r, pl.ds(j * 16, 16))
            a_v[sl] = ga * a_v[sl] + gb * b_v[sl]
        return carry

    lax.fori_loop(0, CHUNK, body, 0)
    pltpu.sync_copy(a_v, y_hbm.at[pl.ds(base, CHUNK)])


@functools.lru_cache(maxsize=None)
def _sc_kernels():
    mesh = plsc.VectorSubcoreMesh(core_axis_name="c", subcore_axis_name="s")
    dispatch = functools.partial(
        pl.kernel,
        out_type=jax.ShapeDtypeStruct((S, D), jnp.float32),
        mesh=mesh,
        scratch_types=[pltpu.VMEM((CHUNK,), jnp.int32),
                       pltpu.VMEM((CHUNK,), jnp.int32),
                       pltpu.VMEM((CHUNK, D), jnp.float32),
                       pltpu.SemaphoreType.DMA,
                       pltpu.SemaphoreType.DMA],
    )(_dispatch)
    combine = functools.partial(
        pl.kernel,
        out_type=jax.ShapeDtypeStruct((N, O), jnp.float32),
        mesh=mesh,
        scratch_types=[pltpu.VMEM((CHUNK,), jnp.int32),
                       pltpu.VMEM((CHUNK,), jnp.int32),
                       pltpu.VMEM((CHUNK + 16,), jnp.float32),
                       pltpu.VMEM((CHUNK + 16,), jnp.float32),
                       pltpu.VMEM((CHUNK, O), jnp.float32),
                       pltpu.VMEM((CHUNK, O), jnp.float32),
                       pltpu.SemaphoreType.DMA,
                       pltpu.SemaphoreType.DMA],
    )(_combine)
    return dispatch, combine


def _ffn_body(eot_ref, xs_ref, w1_ref, b1_ref, w2_ref, b2_ref, out_ref):
    xb = xs_ref[...].astype(jnp.bfloat16)
    w1 = w1_ref[0].astype(jnp.bfloat16)
    w2 = w2_ref[0].astype(jnp.bfloat16)
    h = jnp.dot(xb, w1, preferred_element_type=jnp.float32)
    h = jnp.maximum(h + b1_ref[0], 0.0).astype(jnp.bfloat16)
    out_ref[...] = jnp.dot(h, w2,
                           preferred_element_type=jnp.float32) + b2_ref[0]


_ffn_grid = pltpu.PrefetchScalarGridSpec(
    num_scalar_prefetch=1,
    grid=(T,),
    in_specs=[
        pl.BlockSpec((TILE, D), lambda t, eot: (t, 0)),
        pl.BlockSpec((1, D, H), lambda t, eot: (eot[t], 0, 0)),
        pl.BlockSpec((1, 1, H), lambda t, eot: (eot[t], 0, 0)),
        pl.BlockSpec((1, H, O), lambda t, eot: (eot[t], 0, 0)),
        pl.BlockSpec((1, 1, O), lambda t, eot: (eot[t], 0, 0)),
    ],
    out_specs=pl.BlockSpec((TILE, O), lambda t, eot: (t, 0)),
)


def kernel(x, w_gate, w_noise, W1, b1, W2, b2):
    del w_noise
    wg_pad = jnp.zeros((D, LANES), jnp.float32).at[:, :E].set(w_gate)

    (g0o, g1o, pos0o, pos1o, eoto, losso) = pl.pallas_call(
        _router_body,
        grid=(NB,),
        in_specs=[pl.BlockSpec((TB, D), lambda i: (i, 0)),
                  pl.BlockSpec((D, LANES), lambda i: (0, 0))],
        out_specs=[pl.BlockSpec((1, 1, LANES), lambda i: (i, 0, 0))] * 2
        + [pl.BlockSpec((NB, 1, LANES), lambda i: (0, 0, 0))] * 2
        + [pl.BlockSpec((1, 1, LANES), lambda i: (0, 0, 0))] * 2,
        out_shape=[jax.ShapeDtypeStruct((NB, 1, LANES), jnp.float32)] * 2
        + [jax.ShapeDtypeStruct((NB, 1, LANES), jnp.int32)] * 2
        + [jax.ShapeDtypeStruct((1, 1, LANES), jnp.int32)]
        + [jax.ShapeDtypeStruct((1, 1, LANES), jnp.float32)],
        scratch_shapes=[pltpu.VMEM((1, LANES), jnp.float32)] * 3
        + [pltpu.VMEM((NB, LANES), jnp.float32)] * 4,
    )(x, wg_pad)

    g0 = g0o.reshape(N)
    g1 = g1o.reshape(N)
    eot = eoto.reshape(LANES)
    loss = losso.reshape(LANES)[0]
    pos0 = pos0o.reshape(N)
    pos1 = pos1o.reshape(N)

    dispatch, combine = _sc_kernels()
    x_sorted = dispatch(pos0, pos1, x)

    out_sorted = pl.pallas_call(
        _ffn_body,
        grid_spec=_ffn_grid,
        out_shape=jax.ShapeDtypeStruct((S, O), jnp.float32),
    )(eot, x_sorted, W1, b1.reshape(E, 1, H), W2, b2.reshape(E, 1, O))

    y = combine(out_sorted, pos0, pos1, g0, g1)
    return y, loss
